# baseline (speedup 1.0000x reference)
import functools

import jax
import jax.numpy as jnp
from jax.experimental import pallas as pl
from jax.experimental.pallas import tpu as pltpu

B, N, D = 16, 2048, 2048
ROWS = B * N
BLK = 2048


def _matvec_body(x_ref, w_ref, o_ref):
    o_ref[...] = jax.lax.dot_general(
        x_ref[...], w_ref[...],
        dimension_numbers=(((1,), (0,)), ((), ())),
        preferred_element_type=jnp.float32,
    )


def _head_body(s_ref, w1t_ref, b1_ref, w2t_ref, b2_ref, w3t_ref, b3_ref, o_ref):
    s = s_ref[...]
    iota = jax.lax.broadcasted_iota(jnp.int32, (B, N), 1)

    max1 = jnp.max(s, axis=1, keepdims=True)
    idx_max = jnp.min(jnp.where(s == max1, iota, N), axis=1, keepdims=True)
    max2 = jnp.max(jnp.where(iota == idx_max, -jnp.inf, s), axis=1, keepdims=True)

    min1 = jnp.min(s, axis=1, keepdims=True)
    idx_min = jnp.min(jnp.where(s == min1, iota, N), axis=1, keepdims=True)
    min2 = jnp.min(jnp.where(iota == idx_min, jnp.inf, s), axis=1, keepdims=True)

    f = jnp.concatenate([min1, min2, max1, max2], axis=1)

    h = b1_ref[...]
    w1t = w1t_ref[...]
    for k in range(4):
        h = h + f[:, k:k + 1] * w1t[k:k + 1, :]
    h = jax.nn.sigmoid(h)

    h2 = jax.nn.sigmoid(
        jax.lax.dot_general(h, w2t_ref[...],
                            dimension_numbers=(((1,), (0,)), ((), ())),
                            preferred_element_type=jnp.float32)
        + b2_ref[...])

    o_ref[...] = jax.nn.sigmoid(
        jax.lax.dot_general(h2, w3t_ref[...],
                            dimension_numbers=(((1,), (0,)), ((), ())),
                            preferred_element_type=jnp.float32)
        + b3_ref[...])


@jax.jit
def kernel(x, W_embed, W1, b1, W2, b2, W3, b3):
    xf = x.reshape(ROWS, D)
    wt = W_embed.reshape(D, 1)

    scores = pl.pallas_call(
        _matvec_body,
        grid=(ROWS // BLK,),
        in_specs=[
            pl.BlockSpec((BLK, D), lambda i: (i, 0)),
            pl.BlockSpec((D, 1), lambda i: (0, 0)),
        ],
        out_specs=pl.BlockSpec((BLK, 1), lambda i: (i, 0)),
        out_shape=jax.ShapeDtypeStruct((ROWS, 1), jnp.float32),
    )(xf, wt)

    out = pl.pallas_call(
        _head_body,
        out_shape=jax.ShapeDtypeStruct((B, 1), jnp.float32),
    )(
        scores.reshape(B, N),
        W1.T, b1.reshape(1, 200),
        W2.T, b2.reshape(1, 100),
        W3.T, b3.reshape(1, 1),
    )
    return out.reshape(-1)


# speedup vs baseline: 1.0625x; 1.0625x over previous
import jax
import jax.numpy as jnp
from jax.experimental import pallas as pl

B, N, D = 16, 2048, 2048


def _body(x_ref, w_ref, w1t_ref, b1_ref, w2t_ref, b2_ref, w3t_ref, b3_ref,
          o_ref):
    s = jax.lax.dot_general(
        x_ref[...], w_ref[...],
        dimension_numbers=(((1,), (0,)), ((), ())),
        preferred_element_type=jnp.float32,
    )
    iota = jax.lax.broadcasted_iota(jnp.int32, (N, 1), 0)

    max1 = jnp.max(s)
    idx_max = jnp.min(jnp.where(s == max1, iota, N))
    max2 = jnp.max(jnp.where(iota == idx_max, -jnp.inf, s))

    min1 = jnp.min(s)
    idx_min = jnp.min(jnp.where(s == min1, iota, N))
    min2 = jnp.min(jnp.where(iota == idx_min, jnp.inf, s))

    h = (b1_ref[...]
         + min1 * w1t_ref[0:1, :]
         + min2 * w1t_ref[1:2, :]
         + max1 * w1t_ref[2:3, :]
         + max2 * w1t_ref[3:4, :])
    h = jax.nn.sigmoid(h)

    h2 = jax.nn.sigmoid(
        jax.lax.dot_general(h, w2t_ref[...],
                            dimension_numbers=(((1,), (0,)), ((), ())),
                            preferred_element_type=jnp.float32)
        + b2_ref[...])

    i = pl.program_id(0)
    o_ref[pl.ds(i, 1), :] = jax.nn.sigmoid(
        jax.lax.dot_general(h2, w3t_ref[...],
                            dimension_numbers=(((1,), (0,)), ((), ())),
                            preferred_element_type=jnp.float32)
        + b3_ref[...])


@jax.jit
def kernel(x, W_embed, W1, b1, W2, b2, W3, b3):
    xf = x.reshape(B * N, D)
    wt = W_embed.reshape(D, 1)
    const = lambda i: (0, 0)

    out = pl.pallas_call(
        _body,
        grid=(B,),
        in_specs=[
            pl.BlockSpec((N, D), lambda i: (i, 0)),
            pl.BlockSpec((D, 1), const),
            pl.BlockSpec((4, 200), const),
            pl.BlockSpec((1, 200), const),
            pl.BlockSpec((200, 100), const),
            pl.BlockSpec((1, 100), const),
            pl.BlockSpec((100, 1), const),
            pl.BlockSpec((1, 1), const),
        ],
        out_specs=pl.BlockSpec((B, 1), lambda i: (0, 0)),
        out_shape=jax.ShapeDtypeStruct((B, 1), jnp.float32),
    )(xf, wt, W1.T, b1.reshape(1, 200), W2.T, b2.reshape(1, 100),
      W3.T, b3.reshape(1, 1))
    return out.reshape(-1)
